# baseline (speedup 1.0000x reference)
import jax
import jax.numpy as jnp
from jax.experimental import pallas as pl
from jax.experimental.pallas import tpu as pltpu

N = 100000
D_IN = 128
D_HID = 256
D_OUT = 128
NSTREAM = 4
BLOCK = 5000
NBUF = 2
K_OUT = 2
STEPS = N // (NSTREAM * BLOCK)


def _outer(x_hbm, w1_ref, b1_ref, w2_ref, b2_ref, out_hbm, obuf, osem):
    w1v = w1_ref[...]
    w2b = w2_ref[...].astype(jnp.bfloat16)
    b1v = b1_ref[...]
    b2v = b2_ref[...]

    def _copy(s, step, slot):
        return pltpu.make_async_copy(
            obuf.at[s, slot],
            out_hbm.at[pl.ds((s * STEPS + step) * BLOCK, BLOCK), :],
            osem.at[s, slot],
        )

    def _mlp(x_f32):
        h = jnp.dot(x_f32, w1v, preferred_element_type=jnp.float32)
        h = jnp.maximum(h + b1v, 0.0).astype(jnp.bfloat16)
        out = jnp.dot(h, w2b, preferred_element_type=jnp.float32)
        return out + b2v

    def inner(idxs, *x_refs):
        i = idxs[0]
        slot = jax.lax.rem(i, K_OUT)

        @pl.when(i >= K_OUT)
        def _wait_prev():
            for s in range(NSTREAM):
                _copy(s, i - K_OUT, slot).wait()

        for s in range(NSTREAM):
            obuf[s, slot] = _mlp(x_refs[s][...])
            _copy(s, i, slot).start()

    def _spec(s):
        return pl.BlockSpec(
            (BLOCK, D_IN), lambda i, s=s: (s * STEPS + i, 0),
            pipeline_mode=pl.Buffered(buffer_count=NBUF),
        )

    pltpu.emit_pipeline(
        inner,
        grid=(STEPS,),
        in_specs=[_spec(s) for s in range(NSTREAM)],
        out_specs=[],
        _explicit_indices=True,
    )(*([x_hbm] * NSTREAM))

    for j in range(max(0, STEPS - K_OUT), STEPS):
        for s in range(NSTREAM):
            _copy(s, j, j % K_OUT).wait()


@jax.jit
def kernel(x, W1, b1, W2, b2):
    b1r = b1.reshape(1, D_HID)
    b2r = b2.reshape(1, D_OUT)
    return pl.pallas_call(
        _outer,
        in_specs=[
            pl.BlockSpec(memory_space=pltpu.MemorySpace.HBM),
            pl.BlockSpec(memory_space=pltpu.MemorySpace.VMEM),
            pl.BlockSpec(memory_space=pltpu.MemorySpace.VMEM),
            pl.BlockSpec(memory_space=pltpu.MemorySpace.VMEM),
            pl.BlockSpec(memory_space=pltpu.MemorySpace.VMEM),
        ],
        out_specs=pl.BlockSpec(memory_space=pltpu.MemorySpace.HBM),
        out_shape=jax.ShapeDtypeStruct((N, D_OUT), jnp.float32),
        scratch_shapes=[
            pltpu.VMEM((NSTREAM, K_OUT, BLOCK, D_OUT), jnp.float32),
            pltpu.SemaphoreType.DMA((NSTREAM, K_OUT)),
        ],
    )(x, W1, b1r, W2, b2r)


# speedup vs baseline: 1.0767x; 1.0767x over previous
import jax
import jax.numpy as jnp
from jax.experimental import pallas as pl
from jax.experimental.pallas import tpu as pltpu

N = 100000
D_IN = 128
D_HID = 256
D_OUT = 128
BLOCK = 5000
NBUF = 6
K_OUT = 4
STEPS = N // (2 * BLOCK)


def _outer(x_hbm, w1_ref, b1_ref, w2_ref, b2_ref, out_hbm, obuf_a, obuf_b, osem):
    w1b = w1_ref[...].astype(jnp.bfloat16)
    w2b = w2_ref[...].astype(jnp.bfloat16)
    b1v = b1_ref[...]
    b2v = b2_ref[...]

    def _copy_a(step, slot):
        return pltpu.make_async_copy(
            obuf_a.at[slot],
            out_hbm.at[pl.ds(step * BLOCK, BLOCK), :],
            osem.at[0, slot],
        )

    def _copy_b(step, slot):
        return pltpu.make_async_copy(
            obuf_b.at[slot],
            out_hbm.at[pl.ds((STEPS + step) * BLOCK, BLOCK), :],
            osem.at[1, slot],
        )

    def _mlp(x_f32):
        h = jnp.dot(x_f32, w1_ref[...], preferred_element_type=jnp.float32)
        h = jnp.maximum(h + b1v, 0.0).astype(jnp.bfloat16)
        out = jnp.dot(h, w2b, preferred_element_type=jnp.float32)
        return out + b2v

    def inner(idxs, xa_ref, xb_ref):
        i = idxs[0]
        slot = jax.lax.rem(i, K_OUT)

        @pl.when(i >= K_OUT)
        def _wait_prev():
            _copy_a(i - K_OUT, slot).wait()
            _copy_b(i - K_OUT, slot).wait()

        obuf_a[slot] = _mlp(xa_ref[...])
        _copy_a(i, slot).start()
        obuf_b[slot] = _mlp(xb_ref[...])
        _copy_b(i, slot).start()

    pltpu.emit_pipeline(
        inner,
        grid=(STEPS,),
        in_specs=[
            pl.BlockSpec(
                (BLOCK, D_IN), lambda i: (i, 0),
                pipeline_mode=pl.Buffered(buffer_count=NBUF),
            ),
            pl.BlockSpec(
                (BLOCK, D_IN), lambda i: (STEPS + i, 0),
                pipeline_mode=pl.Buffered(buffer_count=NBUF),
            ),
        ],
        out_specs=[],
        _explicit_indices=True,
    )(x_hbm, x_hbm)

    for j in range(max(0, STEPS - K_OUT), STEPS):
        _copy_a(j, j % K_OUT).wait()
        _copy_b(j, j % K_OUT).wait()


@jax.jit
def kernel(x, W1, b1, W2, b2):
    b1r = b1.reshape(1, D_HID)
    b2r = b2.reshape(1, D_OUT)
    return pl.pallas_call(
        _outer,
        in_specs=[
            pl.BlockSpec(memory_space=pltpu.MemorySpace.HBM),
            pl.BlockSpec(memory_space=pltpu.MemorySpace.VMEM),
            pl.BlockSpec(memory_space=pltpu.MemorySpace.VMEM),
            pl.BlockSpec(memory_space=pltpu.MemorySpace.VMEM),
            pl.BlockSpec(memory_space=pltpu.MemorySpace.VMEM),
        ],
        out_specs=pl.BlockSpec(memory_space=pltpu.MemorySpace.HBM),
        out_shape=jax.ShapeDtypeStruct((N, D_OUT), jnp.float32),
        scratch_shapes=[
            pltpu.VMEM((K_OUT, BLOCK, D_OUT), jnp.float32),
            pltpu.VMEM((K_OUT, BLOCK, D_OUT), jnp.float32),
            pltpu.SemaphoreType.DMA((2, K_OUT)),
        ],
    )(x, W1, b1r, W2, b2r)


# speedup vs baseline: 1.0832x; 1.0060x over previous
import jax
import jax.numpy as jnp
from jax.experimental import pallas as pl
from jax.experimental.pallas import tpu as pltpu

N = 100000
D_IN = 128
D_HID = 256
D_OUT = 128
BLOCK = 5000
NBUF = 6
K_OUT = 4
STEPS = N // (2 * BLOCK)


def _outer(x_hbm, w1_ref, b1_ref, w2_ref, b2_ref, out_hbm, obuf_a, obuf_b, osem):
    w2b = w2_ref[...].astype(jnp.bfloat16)
    b1v = b1_ref[...]
    b2v = b2_ref[...]

    def _copy_a(step, slot):
        return pltpu.make_async_copy(
            obuf_a.at[slot],
            out_hbm.at[pl.ds(step * BLOCK, BLOCK), :],
            osem.at[0, slot],
        )

    def _copy_b(step, slot):
        return pltpu.make_async_copy(
            obuf_b.at[slot],
            out_hbm.at[pl.ds((STEPS + step) * BLOCK, BLOCK), :],
            osem.at[1, slot],
        )

    def _mlp(x_f32):
        h = jnp.dot(x_f32, w1_ref[...], preferred_element_type=jnp.float32)
        h = jnp.maximum(h + b1v, 0.0).astype(jnp.bfloat16)
        out = jnp.dot(h, w2b, preferred_element_type=jnp.float32)
        return out + b2v

    def inner(idxs, xa_ref, xb_ref):
        i = idxs[0]
        slot = jax.lax.rem(i, K_OUT)

        @pl.when(i >= K_OUT)
        def _wait_prev():
            _copy_a(i - K_OUT, slot).wait()
            _copy_b(i - K_OUT, slot).wait()

        obuf_a[slot] = _mlp(xa_ref[...])
        _copy_a(i, slot).start()
        obuf_b[slot] = _mlp(xb_ref[...])
        _copy_b(i, slot).start()

    pltpu.emit_pipeline(
        inner,
        grid=(STEPS,),
        in_specs=[
            pl.BlockSpec(
                (BLOCK, D_IN), lambda i: (i, 0),
                pipeline_mode=pl.Buffered(buffer_count=NBUF),
            ),
            pl.BlockSpec(
                (BLOCK, D_IN), lambda i: (STEPS + i, 0),
                pipeline_mode=pl.Buffered(buffer_count=NBUF),
            ),
        ],
        out_specs=[],
        _explicit_indices=True,
    )(x_hbm, x_hbm)

    for j in range(max(0, STEPS - K_OUT), STEPS):
        _copy_a(j, j % K_OUT).wait()
        _copy_b(j, j % K_OUT).wait()


@jax.jit
def kernel(x, W1, b1, W2, b2):
    b1r = b1.reshape(1, D_HID)
    b2r = b2.reshape(1, D_OUT)
    return pl.pallas_call(
        _outer,
        in_specs=[
            pl.BlockSpec(memory_space=pltpu.MemorySpace.HBM),
            pl.BlockSpec(memory_space=pltpu.MemorySpace.VMEM),
            pl.BlockSpec(memory_space=pltpu.MemorySpace.VMEM),
            pl.BlockSpec(memory_space=pltpu.MemorySpace.VMEM),
            pl.BlockSpec(memory_space=pltpu.MemorySpace.VMEM),
        ],
        out_specs=pl.BlockSpec(memory_space=pltpu.MemorySpace.HBM),
        out_shape=jax.ShapeDtypeStruct((N, D_OUT), jnp.float32),
        scratch_shapes=[
            pltpu.VMEM((K_OUT, BLOCK, D_OUT), jnp.float32),
            pltpu.VMEM((K_OUT, BLOCK, D_OUT), jnp.float32),
            pltpu.SemaphoreType.DMA((2, K_OUT)),
        ],
    )(x, W1, b1r, W2, b2r)
